# Initial kernel scaffold; baseline (speedup 1.0000x reference)
import jax
import jax.numpy as jnp
from jax.experimental import pallas as pl


def kernel(node_input, edge_attr, edge_scalars, edge_src, edge_dst, W_exp, b_exp, W1, b1, ln_g, ln_b, W2, offset, W_proj, b_proj):
    raise NotImplementedError("write your pallas kernel here")


# speedup vs baseline: 28.0019x; 28.0019x over previous
import functools

import jax
import jax.numpy as jnp
from jax import lax
from jax.experimental import pallas as pl
from jax.experimental.pallas import tpu as pltpu
from jax.experimental.pallas import tpu_sc as plsc

N = 10000
E = 160000
C = 128
FC = 64

NC = 2
NS = 16

EPT = E // NS
CH = 80
NCH = EPT // CH
RPT = ((-(-N // NS) + 7) // 8) * 8
NP = RPT * NS

BE = 1280
NB = E // BE


def _tc_body(es_ref, attr_ref, w1t_ref, b1_ref, lng_ref, lnb_ref,
             w2ct_ref, offc_ref, bias_ref, y_ref):
    h = jnp.dot(es_ref[...], w1t_ref[...],
                preferred_element_type=jnp.float32) + b1_ref[...]
    m = jnp.mean(h, axis=1, keepdims=True)
    hc = h - m
    v = jnp.mean(hc * hc, axis=1, keepdims=True)
    h = hc * lax.rsqrt(v + 1e-5) * lng_ref[...] + lnb_ref[...]
    h = h * jax.nn.sigmoid(h)
    g = jnp.dot(h, w2ct_ref[...], preferred_element_type=jnp.float32) + offc_ref[...]
    a = g[:, :C]
    b = g[:, C:]
    attr = attr_ref[...]
    y_ref[0] = attr[:, 0:1] * a + bias_ref[...]
    y_ref[1] = attr[:, 1:2] * b
    y_ref[2] = attr[:, 2:3] * b
    y_ref[3] = attr[:, 3:4] * b


def _tc_stage(es, attr, w1t, b1, lng, lnb, w2ct, offc, bias):
    wspec = pl.BlockSpec((w1t.shape[0], w1t.shape[1]), lambda i: (0, 0))
    return pl.pallas_call(
        _tc_body,
        grid=(NB,),
        in_specs=[
            pl.BlockSpec((BE, FC), lambda i: (i, 0)),
            pl.BlockSpec((BE, 4), lambda i: (i, 0)),
            wspec,
            pl.BlockSpec((1, FC), lambda i: (0, 0)),
            pl.BlockSpec((1, FC), lambda i: (0, 0)),
            pl.BlockSpec((1, FC), lambda i: (0, 0)),
            pl.BlockSpec((FC, 2 * C), lambda i: (0, 0)),
            pl.BlockSpec((1, 2 * C), lambda i: (0, 0)),
            pl.BlockSpec((1, C), lambda i: (0, 0)),
        ],
        out_specs=pl.BlockSpec((4, BE, C), lambda i: (0, i, 0)),
        out_shape=jax.ShapeDtypeStruct((4, E, C), jnp.float32),
        compiler_params=pltpu.CompilerParams(
            dimension_semantics=("arbitrary",)),
    )(es, attr, w1t, b1, lng, lnb, w2ct, offc, bias)


def _sc_body(y_hbm, dst_hbm, z_hbm, out_hbm,
             acc, yb0, yb1, db0, db1, sy0, sy1, sd0, sd1):
    c = lax.axis_index("c")
    s = lax.axis_index("s")
    ebase = s * EPT
    rbase = s * RPT

    for j in range(2):
        comp = 2 * c + j
        row0 = comp * E + ebase

        pltpu.sync_copy(z_hbm, acc.at[pl.ds(rbase, RPT)])
        plsc.subcore_barrier()

        def load(k, yb, db, sy, sd):
            pltpu.async_copy(y_hbm.at[pl.ds(row0 + k * CH, CH)], yb, sy)
            pltpu.async_copy(dst_hbm.at[pl.ds(ebase + k * CH, CH)], db, sd)

        def waitload(yb, db, sy, sd):
            pltpu.make_async_copy(y_hbm.at[pl.ds(row0, CH)], yb, sy).wait()
            pltpu.make_async_copy(dst_hbm.at[pl.ds(ebase, CH)], db, sd).wait()

        load(0, yb0, db0, sy0, sd0)

        def body(kk, carry):
            k1 = 2 * kk + 1
            waitload(yb0, db0, sy0, sd0)

            @pl.when(k1 < NCH)
            def _():
                load(k1, yb1, db1, sy1, sd1)

            pltpu.sync_copy(yb0, acc.at[db0], add=True)

            @pl.when(k1 < NCH)
            def _():
                waitload(yb1, db1, sy1, sd1)

                @pl.when(k1 + 1 < NCH)
                def _():
                    load(k1 + 1, yb0, db0, sy0, sd0)

                pltpu.sync_copy(yb1, acc.at[db1], add=True)

            return carry

        lax.fori_loop(0, (NCH + 1) // 2, body, 0)
        plsc.subcore_barrier()

        pltpu.sync_copy(acc.at[pl.ds(rbase, RPT)],
                        out_hbm.at[pl.ds(comp * NP + rbase, RPT)])


def _sc_scatter_call():
    return functools.partial(
        pl.kernel,
        out_type=jax.ShapeDtypeStruct((4 * NP, C), jnp.float32),
        mesh=plsc.VectorSubcoreMesh(
            core_axis_name="c", subcore_axis_name="s",
            num_cores=NC, num_subcores=NS),
        scratch_types=[
            pltpu.VMEM_SHARED((NP, C), jnp.float32),
            pltpu.VMEM((CH, C), jnp.float32),
            pltpu.VMEM((CH, C), jnp.float32),
            pltpu.VMEM((CH,), jnp.int32),
            pltpu.VMEM((CH,), jnp.int32),
            pltpu.SemaphoreType.DMA,
            pltpu.SemaphoreType.DMA,
            pltpu.SemaphoreType.DMA,
            pltpu.SemaphoreType.DMA,
        ],
    )(_sc_body)


def kernel(node_input, edge_attr, edge_scalars, edge_src, edge_dst,
           W_exp, b_exp, W1, b1, ln_g, ln_b, W2, offset, W_proj, b_proj):
    s = W_exp.sum(axis=1) + b_exp[0]
    Wp = W_proj * s[None, :]
    w2c = jnp.concatenate([Wp @ W2[:C], Wp @ W2[C:]], axis=0) * 0.25
    offc = jnp.concatenate([Wp @ offset[:C], Wp @ offset[C:]]) * 0.25
    bias = b_proj[0] * 0.25

    y = _tc_stage(edge_scalars, edge_attr, W1.T, b1[None], ln_g[None],
                  ln_b[None], w2c.T, offc[None], bias[None])

    out = _sc_scatter_call()(y.reshape(4 * E, C), edge_dst,
                             jnp.zeros((RPT, C), jnp.float32))
    return out.reshape(4, NP, C)[:, :N].transpose(1, 0, 2)


# speedup vs baseline: 36.0827x; 1.2886x over previous
import functools

import jax
import jax.numpy as jnp
from jax import lax
from jax.experimental import pallas as pl
from jax.experimental.pallas import tpu as pltpu
from jax.experimental.pallas import tpu_sc as plsc

N = 10000
E = 160000
C = 128
FC = 64

NC = 2
NS = 16

EPT = E // NS
CH = 80
NCH = EPT // CH
RPT = ((-(-N // NS) + 7) // 8) * 8

BE = 1280
NB = E // BE


def _tc_body(esT_ref, attrT_ref, sel_ref, w1_ref, b1_ref, lng_ref, lnb_ref,
             w2ct_ref, offc_ref, bias_ref, y_ref):
    h = lax.dot_general(esT_ref[...], w1_ref[...],
                        dimension_numbers=(((0,), (1,)), ((), ())),
                        preferred_element_type=jnp.float32) + b1_ref[...]
    m = jnp.mean(h, axis=1, keepdims=True)
    hc = h - m
    v = jnp.mean(hc * hc, axis=1, keepdims=True)
    h = hc * lax.rsqrt(v + 1e-5) * lng_ref[...] + lnb_ref[...]
    h = h * jax.nn.sigmoid(h)
    g = jnp.dot(h, w2ct_ref[...], preferred_element_type=jnp.float32) + offc_ref[...]
    sc = lax.dot_general(attrT_ref[...], sel_ref[...],
                         dimension_numbers=(((0,), (0,)), ((), ())),
                         preferred_element_type=jnp.float32)
    a = g[:, :C]
    b = g[:, C:]
    y_ref[0] = sc[:, 0:C] * a + bias_ref[...]
    y_ref[1] = sc[:, C:2 * C] * b
    y_ref[2] = sc[:, 2 * C:3 * C] * b
    y_ref[3] = sc[:, 3 * C:] * b


def _tc_stage(esT, attrT, sel, w1, b1, lng, lnb, w2ct, offc, bias):
    return pl.pallas_call(
        _tc_body,
        grid=(NB,),
        in_specs=[
            pl.BlockSpec((FC, BE), lambda i: (0, i)),
            pl.BlockSpec((4, BE), lambda i: (0, i)),
            pl.BlockSpec((4, 4 * C), lambda i: (0, 0)),
            pl.BlockSpec((FC, FC), lambda i: (0, 0)),
            pl.BlockSpec((1, FC), lambda i: (0, 0)),
            pl.BlockSpec((1, FC), lambda i: (0, 0)),
            pl.BlockSpec((1, FC), lambda i: (0, 0)),
            pl.BlockSpec((FC, 2 * C), lambda i: (0, 0)),
            pl.BlockSpec((1, 2 * C), lambda i: (0, 0)),
            pl.BlockSpec((1, C), lambda i: (0, 0)),
        ],
        out_specs=pl.BlockSpec((4, BE, C), lambda i: (0, i, 0)),
        out_shape=jax.ShapeDtypeStruct((4, E, C), jnp.float32),
        compiler_params=pltpu.CompilerParams(
            dimension_semantics=("arbitrary",)),
    )(esT, attrT, sel, w1, b1, lng, lnb, w2ct, offc, bias)


def _sc_body(y_hbm, dst_hbm, z_hbm, out_hbm,
             acc, yb0, yb1, db0, db1, sy0, sy1, sd0, sd1):
    c = lax.axis_index("c")
    s = lax.axis_index("s")
    ebase = s * EPT
    nbase = jnp.minimum(s * RPT, N - RPT)

    for j in range(2):
        comp = 2 * c + j
        row0 = comp * E + ebase

        pltpu.sync_copy(z_hbm, acc.at[pl.ds(nbase, RPT)])
        plsc.subcore_barrier()

        def load(k, yb, db, sy, sd):
            pltpu.async_copy(y_hbm.at[pl.ds(row0 + k * CH, CH)], yb, sy)
            pltpu.async_copy(dst_hbm.at[pl.ds(ebase + k * CH, CH)], db, sd)

        def waitload(yb, db, sy, sd):
            pltpu.make_async_copy(y_hbm.at[pl.ds(row0, CH)], yb, sy).wait()
            pltpu.make_async_copy(dst_hbm.at[pl.ds(ebase, CH)], db, sd).wait()

        load(0, yb0, db0, sy0, sd0)

        def body(kk, carry):
            k1 = 2 * kk + 1
            waitload(yb0, db0, sy0, sd0)

            @pl.when(k1 < NCH)
            def _():
                load(k1, yb1, db1, sy1, sd1)

            pltpu.sync_copy(yb0, acc.at[db0], add=True)

            @pl.when(k1 < NCH)
            def _():
                waitload(yb1, db1, sy1, sd1)

                @pl.when(k1 + 1 < NCH)
                def _():
                    load(k1 + 1, yb0, db0, sy0, sd0)

                pltpu.sync_copy(yb1, acc.at[db1], add=True)

            return carry

        lax.fori_loop(0, (NCH + 1) // 2, body, 0)
        plsc.subcore_barrier()

        pltpu.sync_copy(acc.at[pl.ds(nbase, RPT)],
                        out_hbm.at[pl.ds(nbase, RPT), comp, :])


def _sc_scatter_call():
    return functools.partial(
        pl.kernel,
        out_type=jax.ShapeDtypeStruct((N, 4, C), jnp.float32),
        mesh=plsc.VectorSubcoreMesh(
            core_axis_name="c", subcore_axis_name="s",
            num_cores=NC, num_subcores=NS),
        scratch_types=[
            pltpu.VMEM_SHARED((N, C), jnp.float32),
            pltpu.VMEM((CH, C), jnp.float32),
            pltpu.VMEM((CH, C), jnp.float32),
            pltpu.VMEM((CH,), jnp.int32),
            pltpu.VMEM((CH,), jnp.int32),
            pltpu.SemaphoreType.DMA,
            pltpu.SemaphoreType.DMA,
            pltpu.SemaphoreType.DMA,
            pltpu.SemaphoreType.DMA,
        ],
    )(_sc_body)


def kernel(node_input, edge_attr, edge_scalars, edge_src, edge_dst,
           W_exp, b_exp, W1, b1, ln_g, ln_b, W2, offset, W_proj, b_proj):
    s = W_exp.sum(axis=1) + b_exp[0]
    Wp = W_proj * s[None, :]
    w2c = jnp.concatenate([Wp @ W2[:C], Wp @ W2[C:]], axis=0) * 0.25
    offc = jnp.concatenate([Wp @ offset[:C], Wp @ offset[C:]]) * 0.25
    bias = b_proj[0] * 0.25
    sel = jnp.kron(jnp.eye(4, dtype=jnp.float32), jnp.ones((1, C), jnp.float32))

    y = _tc_stage(edge_scalars.T, edge_attr.T, sel, W1, b1[None], ln_g[None],
                  ln_b[None], w2c.T, offc[None], bias[None])

    return _sc_scatter_call()(y.reshape(4 * E, C), edge_dst,
                              jnp.zeros((RPT, C), jnp.float32))


# speedup vs baseline: 42.8026x; 1.1862x over previous
import functools

import jax
import jax.numpy as jnp
from jax import lax
from jax.experimental import pallas as pl
from jax.experimental.pallas import tpu as pltpu
from jax.experimental.pallas import tpu_sc as plsc

N = 10000
E = 160000
C = 128
FC = 64

NC = 2
NS = 16

EPT = E // NS
CH = 80
NCH = EPT // CH
NSLOT = 4
RPT = ((-(-N // NS) + 7) // 8) * 8

BE = 1280
NB = E // BE


def _tc_body(esT_ref, attrT_ref, sel_ref, w1_ref, b1_ref, lng_ref, lnb_ref,
             w2ct_ref, offc_ref, bias_ref, y_ref):
    h = lax.dot_general(esT_ref[...], w1_ref[...],
                        dimension_numbers=(((0,), (1,)), ((), ())),
                        preferred_element_type=jnp.float32) + b1_ref[...]
    m = jnp.mean(h, axis=1, keepdims=True)
    hc = h - m
    v = jnp.mean(hc * hc, axis=1, keepdims=True)
    h = hc * lax.rsqrt(v + 1e-5) * lng_ref[...] + lnb_ref[...]
    h = h * jax.nn.sigmoid(h)
    g = jnp.dot(h, w2ct_ref[...], preferred_element_type=jnp.float32) + offc_ref[...]
    sc = lax.dot_general(attrT_ref[...], sel_ref[...],
                         dimension_numbers=(((0,), (0,)), ((), ())),
                         preferred_element_type=jnp.float32)
    a = g[:, :C]
    b = g[:, C:]
    y_ref[0] = sc[:, 0:C] * a + bias_ref[...]
    y_ref[1] = sc[:, C:2 * C] * b
    y_ref[2] = sc[:, 2 * C:3 * C] * b
    y_ref[3] = sc[:, 3 * C:] * b


def _tc_stage(esT, attrT, sel, w1, b1, lng, lnb, w2ct, offc, bias):
    return pl.pallas_call(
        _tc_body,
        grid=(NB,),
        in_specs=[
            pl.BlockSpec((FC, BE), lambda i: (0, i)),
            pl.BlockSpec((4, BE), lambda i: (0, i)),
            pl.BlockSpec((4, 4 * C), lambda i: (0, 0)),
            pl.BlockSpec((FC, FC), lambda i: (0, 0)),
            pl.BlockSpec((1, FC), lambda i: (0, 0)),
            pl.BlockSpec((1, FC), lambda i: (0, 0)),
            pl.BlockSpec((1, FC), lambda i: (0, 0)),
            pl.BlockSpec((FC, 2 * C), lambda i: (0, 0)),
            pl.BlockSpec((1, 2 * C), lambda i: (0, 0)),
            pl.BlockSpec((1, C), lambda i: (0, 0)),
        ],
        out_specs=pl.BlockSpec((4, BE, C), lambda i: (0, i, 0)),
        out_shape=jax.ShapeDtypeStruct((4, E, C), jnp.float32),
        compiler_params=pltpu.CompilerParams(
            dimension_semantics=("arbitrary",)),
    )(esT, attrT, sel, w1, b1, lng, lnb, w2ct, offc, bias)


def _sc_body(y_hbm, dst_hbm, z_hbm, out_hbm,
             acc, yb0, yb1, yb2, yb3, db0, db1, db2, db3,
             sl0, sl1, sl2, sl3, ss0, ss1, ss2, ss3):
    ybs = (yb0, yb1, yb2, yb3)
    dbs = (db0, db1, db2, db3)
    sls = (sl0, sl1, sl2, sl3)
    sss = (ss0, ss1, ss2, ss3)
    c = lax.axis_index("c")
    s = lax.axis_index("s")
    ebase = s * EPT
    nbase = jnp.minimum(s * RPT, N - RPT)

    for j in range(2):
        comp = 2 * c + j
        row0 = comp * E + ebase

        pltpu.sync_copy(z_hbm, acc.at[pl.ds(nbase, RPT)])
        plsc.subcore_barrier()

        def load(k, b):
            pltpu.async_copy(y_hbm.at[pl.ds(row0 + k * CH, CH)],
                             ybs[b], sls[b])
            pltpu.async_copy(dst_hbm.at[pl.ds(ebase + k * CH, CH)],
                             dbs[b], sls[b])

        def waitload(b):
            pltpu.make_async_copy(y_hbm.at[pl.ds(row0, CH)],
                                  ybs[b], sls[b]).wait()
            pltpu.make_async_copy(dst_hbm.at[pl.ds(ebase, CH)],
                                  dbs[b], sls[b]).wait()

        def scat_start(b):
            pltpu.async_copy(ybs[b], acc.at[dbs[b]], sss[b], add=True)

        def scat_wait(b):
            pltpu.make_async_copy(ybs[b], acc.at[dbs[b]], sss[b]).wait()

        for b in range(NSLOT):
            load(b, b)

        def body(kk, carry):
            k0 = NSLOT * kk
            for b in range(NSLOT):
                @pl.when(k0 + b < NCH)
                def _(b=b):
                    waitload(b)
                    scat_start(b)
            for b in range(NSLOT):
                @pl.when(k0 + b < NCH)
                def _(b=b):
                    scat_wait(b)

                    @pl.when(k0 + b + NSLOT < NCH)
                    def _(b=b):
                        load(k0 + NSLOT + b, b)

            return carry

        lax.fori_loop(0, -(-NCH // NSLOT), body, 0)
        plsc.subcore_barrier()

        pltpu.sync_copy(acc.at[pl.ds(nbase, RPT)],
                        out_hbm.at[pl.ds(nbase, RPT), comp, :])


def _sc_scatter_call():
    return functools.partial(
        pl.kernel,
        out_type=jax.ShapeDtypeStruct((N, 4, C), jnp.float32),
        mesh=plsc.VectorSubcoreMesh(
            core_axis_name="c", subcore_axis_name="s",
            num_cores=NC, num_subcores=NS),
        scratch_types=(
            [pltpu.VMEM_SHARED((N, C), jnp.float32)]
            + [pltpu.VMEM((CH, C), jnp.float32)] * NSLOT
            + [pltpu.VMEM((CH,), jnp.int32)] * NSLOT
            + [pltpu.SemaphoreType.DMA] * (2 * NSLOT)
        ),
    )(_sc_body)


def kernel(node_input, edge_attr, edge_scalars, edge_src, edge_dst,
           W_exp, b_exp, W1, b1, ln_g, ln_b, W2, offset, W_proj, b_proj):
    s = W_exp.sum(axis=1) + b_exp[0]
    Wp = W_proj * s[None, :]
    w2c = jnp.concatenate([Wp @ W2[:C], Wp @ W2[C:]], axis=0) * 0.25
    offc = jnp.concatenate([Wp @ offset[:C], Wp @ offset[C:]]) * 0.25
    bias = b_proj[0] * 0.25
    sel = jnp.kron(jnp.eye(4, dtype=jnp.float32), jnp.ones((1, C), jnp.float32))

    y = _tc_stage(edge_scalars.T, edge_attr.T, sel, W1, b1[None], ln_g[None],
                  ln_b[None], w2c.T, offc[None], bias[None])

    return _sc_scatter_call()(y.reshape(4 * E, C), edge_dst,
                              jnp.zeros((RPT, C), jnp.float32))
